# baseline (speedup 1.0000x reference)
import functools

import jax
import jax.numpy as jnp
from jax import lax
from jax.experimental import pallas as pl
from jax.experimental.pallas import tpu as pltpu
from jax.experimental.pallas import tpu_sc as plsc

NC, NS, L = 2, 16, 16
NW = NC * NS
B, A, E = 16384, 20, 64
BW = B // NW
CB = 8
NCH = BW // CB
PR = CB * A
CR = 2 * PR
GSL = 80

@functools.cache
def _build_transe_sc():
    mesh = plsc.VectorSubcoreMesh(
        core_axis_name="c", subcore_axis_name="s",
        num_cores=NC, num_subcores=NS,
    )

    @functools.partial(
        pl.kernel,
        out_type=jax.ShapeDtypeStruct((B, E), jnp.float32),
        mesh=mesh,
        scratch_types=[
            pltpu.VMEM((PR,), jnp.int32),
            pltpu.VMEM((CR,), jnp.int32),
            pltpu.VMEM((PR, E), jnp.float32),
            pltpu.VMEM((CR, E), jnp.float32),
            pltpu.VMEM((BW, E), jnp.float32),
            pltpu.SemaphoreType.DMA,
        ],
        compiler_params=pltpu.CompilerParams(use_tc_tiling_on_sc=False),
    )
    def _transe_sc(pidx_hbm, cidx_hbm, ptab_hbm, ctab_hbm, out_hbm,
                   pidx_v, cidx_v, prow_v, crow_v, out_v, sem):
        _transe_body(pidx_hbm, cidx_hbm, ptab_hbm, ctab_hbm, out_hbm,
                     pidx_v, cidx_v, prow_v, crow_v, out_v, sem)

    return _transe_sc


def _transe_body(pidx_hbm, cidx_hbm, ptab_hbm, ctab_hbm, out_hbm,
                 pidx_v, cidx_v, prow_v, crow_v, out_v, sem):
    wid = lax.axis_index("s") * NC + lax.axis_index("c")
    base = wid * BW

    def chunk_body(ch, carry):
        pbase = base * A + ch * PR
        cbase = base * (2 * A) + ch * CR
        pltpu.sync_copy(pidx_hbm.at[pl.ds(pbase, PR)], pidx_v)
        pltpu.sync_copy(cidx_hbm.at[pl.ds(cbase, CR)], cidx_v)
        copies = []
        for k in range(PR // GSL):
            copies.append(pltpu.async_copy(
                ptab_hbm.at[pidx_v.at[pl.ds(k * GSL, GSL)]],
                prow_v.at[pl.ds(k * GSL, GSL)], sem))
        for k in range(CR // GSL):
            copies.append(pltpu.async_copy(
                ctab_hbm.at[cidx_v.at[pl.ds(k * GSL, GSL)]],
                crow_v.at[pl.ds(k * GSL, GSL)], sem))
        for cp in copies:
            cp.wait()

        for b in range(CB):
            def atom_body(a, accs):
                p_row = b * A + a
                c_row = 2 * p_row
                out = []
                for s in range(E // L):
                    sl = pl.ds(s * L, L)
                    p = prow_v[p_row, sl]
                    h = crow_v[c_row, sl]
                    t = crow_v[c_row + 1, sl]
                    out.append(accs[s] + (p + (h - t)))
                return tuple(out)

            z = jnp.zeros((L,), jnp.float32)
            accs = lax.fori_loop(0, A, atom_body, (z, z, z, z))
            row = ch * CB + b
            for s in range(E // L):
                out_v[row, pl.ds(s * L, L)] = accs[s]
        return carry

    lax.fori_loop(0, NCH, chunk_body, 0)
    pltpu.sync_copy(out_v, out_hbm.at[pl.ds(base, BW)])


def kernel(sub_indices, const_table, pred_table):
    si = sub_indices.astype(jnp.int32)
    pidx = si[:, :, 0].reshape(B * A)
    cidx = si[:, :, 1:].reshape(B * A * 2)
    return _build_transe_sc()(pidx, cidx, pred_table, const_table)


# speedup vs baseline: 1.0203x; 1.0203x over previous
import functools

import jax
import jax.numpy as jnp
from jax import lax
from jax.experimental import pallas as pl
from jax.experimental.pallas import tpu as pltpu
from jax.experimental.pallas import tpu_sc as plsc

NC, NS, L = 2, 16, 16
NW = NC * NS
B, A, E = 16384, 20, 64
BW = B // NW
CB = 8
NCH = BW // CB
PR = CB * A
CR = 2 * PR
SI = 3 * PR
GSL = 80


@functools.cache
def _build_transe_sc():
    mesh = plsc.VectorSubcoreMesh(
        core_axis_name="c", subcore_axis_name="s",
        num_cores=NC, num_subcores=NS,
    )

    @functools.partial(
        pl.kernel,
        out_type=jax.ShapeDtypeStruct((B, E), jnp.float32),
        mesh=mesh,
        scratch_types=[
            pltpu.VMEM((SI,), jnp.int32),
            pltpu.VMEM((PR,), jnp.int32),
            pltpu.VMEM((CR,), jnp.int32),
            pltpu.VMEM((PR, E), jnp.float32),
            pltpu.VMEM((CR, E), jnp.float32),
            pltpu.VMEM((BW, E), jnp.float32),
            pltpu.SemaphoreType.DMA,
        ],
        compiler_params=pltpu.CompilerParams(
            use_tc_tiling_on_sc=False, needs_layout_passes=False),
    )
    def _transe_sc(sub_hbm, ptab_hbm, ctab_hbm, out_hbm,
                   sub_v, pidx_v, cidx_v, prow_v, crow_v, out_v, sem):
        _transe_body(sub_hbm, ptab_hbm, ctab_hbm, out_hbm,
                     sub_v, pidx_v, cidx_v, prow_v, crow_v, out_v, sem)

    return _transe_sc


def _transe_body(sub_hbm, ptab_hbm, ctab_hbm, out_hbm,
                 sub_v, pidx_v, cidx_v, prow_v, crow_v, out_v, sem):
    wid = lax.axis_index("s") * NC + lax.axis_index("c")
    base = wid * BW
    lanes = lax.iota(jnp.int32, L)

    def chunk_body(ch, carry):
        pltpu.sync_copy(sub_hbm.at[pl.ds((base + ch * CB) * (3 * A), SI)],
                        sub_v)
        for i in range(PR // L):
            src = lanes * 3 + (i * 3 * L)
            pidx_v[pl.ds(i * L, L)] = plsc.load_gather(sub_v, [src])
        for i in range(CR // L):
            k = lanes + i * L
            src = (k >> 1) * 3 + 1 + (k & 1)
            cidx_v[pl.ds(i * L, L)] = plsc.load_gather(sub_v, [src])

        copies = []
        for k in range(PR // GSL):
            copies.append(pltpu.async_copy(
                ptab_hbm.at[pidx_v.at[pl.ds(k * GSL, GSL)]],
                prow_v.at[pl.ds(k * GSL, GSL)], sem))
        for k in range(CR // GSL):
            copies.append(pltpu.async_copy(
                ctab_hbm.at[cidx_v.at[pl.ds(k * GSL, GSL)]],
                crow_v.at[pl.ds(k * GSL, GSL)], sem))
        for cp in copies:
            cp.wait()

        for b in range(CB):
            def atom_body(a, accs):
                p_row = b * A + a
                c_row = 2 * p_row
                out = []
                for s in range(E // L):
                    sl = pl.ds(s * L, L)
                    p = prow_v[p_row, sl]
                    h = crow_v[c_row, sl]
                    t = crow_v[c_row + 1, sl]
                    out.append(accs[s] + (p + (h - t)))
                return tuple(out)

            z = jnp.zeros((L,), jnp.float32)
            accs = lax.fori_loop(0, A, atom_body, (z, z, z, z))
            row = ch * CB + b
            for s in range(E // L):
                out_v[row, pl.ds(s * L, L)] = accs[s]
        return carry

    lax.fori_loop(0, NCH, chunk_body, 0)
    pltpu.sync_copy(out_v, out_hbm.at[pl.ds(base, BW)])


def kernel(sub_indices, const_table, pred_table):
    sub_flat = sub_indices.astype(jnp.int32).reshape(B * A * 3)
    return _build_transe_sc()(sub_flat, pred_table, const_table)


# speedup vs baseline: 1.0427x; 1.0220x over previous
import functools

import jax
import jax.numpy as jnp
from jax import lax
from jax.experimental import pallas as pl
from jax.experimental.pallas import tpu as pltpu
from jax.experimental.pallas import tpu_sc as plsc

NC, NS, L = 2, 16, 16
NW = NC * NS
B, A, E = 16384, 20, 64
BW = B // NW
CB = 16
NCH = BW // CB
PR = CB * A
CR = 2 * PR
SI = 3 * PR
GSL = 80


@functools.cache
def _build_transe_sc():
    mesh = plsc.VectorSubcoreMesh(
        core_axis_name="c", subcore_axis_name="s",
        num_cores=NC, num_subcores=NS,
    )

    @functools.partial(
        pl.kernel,
        out_type=jax.ShapeDtypeStruct((B, E), jnp.float32),
        mesh=mesh,
        scratch_types=[
            pltpu.VMEM((SI,), jnp.int32),
            pltpu.VMEM((PR,), jnp.int32),
            pltpu.VMEM((CR,), jnp.int32),
            pltpu.VMEM((PR, E), jnp.float32),
            pltpu.VMEM((CR, E), jnp.float32),
            pltpu.VMEM((BW, E), jnp.float32),
            pltpu.SemaphoreType.DMA,
        ],
        compiler_params=pltpu.CompilerParams(
            use_tc_tiling_on_sc=False, needs_layout_passes=False),
    )
    def _transe_sc(sub_hbm, ptab_hbm, ctab_hbm, out_hbm,
                   sub_v, pidx_v, cidx_v, prow_v, crow_v, out_v, sem):
        _transe_body(sub_hbm, ptab_hbm, ctab_hbm, out_hbm,
                     sub_v, pidx_v, cidx_v, prow_v, crow_v, out_v, sem)

    return _transe_sc


def _transe_body(sub_hbm, ptab_hbm, ctab_hbm, out_hbm,
                 sub_v, pidx_v, cidx_v, prow_v, crow_v, out_v, sem):
    wid = lax.axis_index("s") * NC + lax.axis_index("c")
    base = wid * BW
    lanes = lax.iota(jnp.int32, L)

    def chunk_body(ch, carry):
        pltpu.sync_copy(sub_hbm.at[pl.ds((base + ch * CB) * (3 * A), SI)],
                        sub_v)
        for i in range(PR // L):
            src = lanes * 3 + (i * 3 * L)
            pidx_v[pl.ds(i * L, L)] = plsc.load_gather(sub_v, [src])
        for i in range(CR // L):
            k = lanes + i * L
            src = (k >> 1) * 3 + 1 + (k & 1)
            cidx_v[pl.ds(i * L, L)] = plsc.load_gather(sub_v, [src])

        copies = []
        for k in range(PR // GSL):
            copies.append(pltpu.async_copy(
                ptab_hbm.at[pidx_v.at[pl.ds(k * GSL, GSL)]],
                prow_v.at[pl.ds(k * GSL, GSL)], sem))
        for k in range(CR // GSL):
            copies.append(pltpu.async_copy(
                ctab_hbm.at[cidx_v.at[pl.ds(k * GSL, GSL)]],
                crow_v.at[pl.ds(k * GSL, GSL)], sem))
        for cp in copies:
            cp.wait()

        for b in range(CB):
            def atom_body(a, accs):
                p_row = b * A + a
                c_row = 2 * p_row
                out = []
                for s in range(E // L):
                    sl = pl.ds(s * L, L)
                    p = prow_v[p_row, sl]
                    h = crow_v[c_row, sl]
                    t = crow_v[c_row + 1, sl]
                    out.append(accs[s] + (p + (h - t)))
                return tuple(out)

            z = jnp.zeros((L,), jnp.float32)
            accs = lax.fori_loop(0, A, atom_body, (z, z, z, z))
            row = ch * CB + b
            for s in range(E // L):
                out_v[row, pl.ds(s * L, L)] = accs[s]
        return carry

    lax.fori_loop(0, NCH, chunk_body, 0)
    pltpu.sync_copy(out_v, out_hbm.at[pl.ds(base, BW)])


def kernel(sub_indices, const_table, pred_table):
    sub_flat = sub_indices.astype(jnp.int32).reshape(B * A * 3)
    return _build_transe_sc()(sub_flat, pred_table, const_table)


# speedup vs baseline: 1.1198x; 1.0739x over previous
import functools

import jax
import jax.numpy as jnp
from jax import lax
from jax.experimental import pallas as pl
from jax.experimental.pallas import tpu as pltpu
from jax.experimental.pallas import tpu_sc as plsc

NC, NS, L = 2, 16, 16
NW = NC * NS
B, A, E = 16384, 20, 64
BW = B // NW
CB = 16
NCH = BW // CB
PR = CB * A
CR = 2 * PR
SI = 3 * PR
GSL = 80


def _mesh():
    return plsc.VectorSubcoreMesh(
        core_axis_name="c", subcore_axis_name="s",
        num_cores=NC, num_subcores=NS,
    )


@functools.cache
def _build_const_sc():
    @functools.partial(
        pl.kernel,
        out_type=jax.ShapeDtypeStruct((B, E), jnp.float32),
        mesh=_mesh(),
        scratch_types=[
            pltpu.VMEM((SI,), jnp.int32),
            pltpu.VMEM((CR,), jnp.int32),
            pltpu.VMEM((CR, E), jnp.float32),
            pltpu.VMEM((BW, E), jnp.float32),
            pltpu.SemaphoreType.DMA,
        ],
        compiler_params=pltpu.CompilerParams(
            use_tc_tiling_on_sc=False, needs_layout_passes=False),
    )
    def _const_sc(sub_hbm, ctab_hbm, out_hbm, sub_v, cidx_v, crow_v, out_v,
                  sem):
        wid = lax.axis_index("s") * NC + lax.axis_index("c")
        base = wid * BW
        lanes = lax.iota(jnp.int32, L)

        def chunk_body(ch, carry):
            pltpu.sync_copy(
                sub_hbm.at[pl.ds((base + ch * CB) * (3 * A), SI)], sub_v)
            for i in range(CR // L):
                k = lanes + i * L
                src = (k >> 1) * 3 + 1 + (k & 1)
                cidx_v[pl.ds(i * L, L)] = plsc.load_gather(sub_v, [src])

            copies = []
            for k in range(CR // GSL):
                copies.append(pltpu.async_copy(
                    ctab_hbm.at[cidx_v.at[pl.ds(k * GSL, GSL)]],
                    crow_v.at[pl.ds(k * GSL, GSL)], sem))
            for cp in copies:
                cp.wait()

            for b in range(CB):
                def atom_body(a, accs):
                    c_row = 2 * (b * A + a)
                    out = []
                    for s in range(E // L):
                        sl = pl.ds(s * L, L)
                        h = crow_v[c_row, sl]
                        t = crow_v[c_row + 1, sl]
                        out.append(accs[s] + (h - t))
                    return tuple(out)

                z = jnp.zeros((L,), jnp.float32)
                accs = lax.fori_loop(0, A, atom_body, (z, z, z, z))
                row = ch * CB + b
                for s in range(E // L):
                    out_v[row, pl.ds(s * L, L)] = accs[s]
            return carry

        lax.fori_loop(0, NCH, chunk_body, 0)
        pltpu.sync_copy(out_v, out_hbm.at[pl.ds(base, BW)])

    return _const_sc


@functools.cache
def _build_pred_sc():
    @functools.partial(
        pl.kernel,
        out_type=jax.ShapeDtypeStruct((B, E), jnp.float32),
        mesh=_mesh(),
        scratch_types=[
            pltpu.VMEM((SI,), jnp.int32),
            pltpu.VMEM((PR,), jnp.int32),
            pltpu.VMEM((PR, E), jnp.float32),
            pltpu.VMEM((BW, E), jnp.float32),
            pltpu.SemaphoreType.DMA,
        ],
        compiler_params=pltpu.CompilerParams(
            use_tc_tiling_on_sc=False, needs_layout_passes=False),
    )
    def _pred_sc(sub_hbm, ptab_hbm, csum_hbm, out_hbm, sub_v, pidx_v, prow_v,
                 out_v, sem):
        wid = lax.axis_index("s") * NC + lax.axis_index("c")
        base = wid * BW
        lanes = lax.iota(jnp.int32, L)

        pltpu.sync_copy(csum_hbm.at[pl.ds(base, BW)], out_v)

        def chunk_body(ch, carry):
            pltpu.sync_copy(
                sub_hbm.at[pl.ds((base + ch * CB) * (3 * A), SI)], sub_v)
            for i in range(PR // L):
                src = lanes * 3 + (i * 3 * L)
                pidx_v[pl.ds(i * L, L)] = plsc.load_gather(sub_v, [src])

            copies = []
            for k in range(PR // GSL):
                copies.append(pltpu.async_copy(
                    ptab_hbm.at[pidx_v.at[pl.ds(k * GSL, GSL)]],
                    prow_v.at[pl.ds(k * GSL, GSL)], sem))
            for cp in copies:
                cp.wait()

            for b in range(CB):
                def atom_body(a, accs):
                    p_row = b * A + a
                    out = []
                    for s in range(E // L):
                        sl = pl.ds(s * L, L)
                        out.append(accs[s] + prow_v[p_row, sl])
                    return tuple(out)

                row = ch * CB + b
                init = tuple(out_v[row, pl.ds(s * L, L)]
                             for s in range(E // L))
                accs = lax.fori_loop(0, A, atom_body, init)
                for s in range(E // L):
                    out_v[row, pl.ds(s * L, L)] = accs[s]
            return carry

        lax.fori_loop(0, NCH, chunk_body, 0)
        pltpu.sync_copy(out_v, out_hbm.at[pl.ds(base, BW)])

    return _pred_sc


def kernel(sub_indices, const_table, pred_table):
    sub_flat = sub_indices.astype(jnp.int32).reshape(B * A * 3)
    csum = _build_const_sc()(sub_flat, const_table)
    return _build_pred_sc()(sub_flat, pred_table, csum)
